# Initial kernel scaffold; baseline (speedup 1.0000x reference)
import jax
import jax.numpy as jnp
from jax.experimental import pallas as pl


def kernel(q, k, linear, topk):
    raise NotImplementedError("write your pallas kernel here")


# speedup vs baseline: 6.2927x; 6.2927x over previous
import functools

import jax
import jax.numpy as jnp
from jax.experimental import pallas as pl
import jax.experimental.pallas.tpu as pltpu

HEADS = 32
BUCKET_SIZE = 128
MAX_BUCKETS = 64
N_TOP_BUCKETS = 8
DIM = 256
TEMPERATURE = 0.75


def _sortnet_kernel(k_ref, w_ref, out_ref, *, topk, buckets):
    kb = k_ref[0]
    d_h = kb.shape[-1]
    kb = kb.reshape(buckets, BUCKET_SIZE, d_h)
    firsts = kb[:, 0, :]
    sums = kb[:, 0, :]
    for t in range(1, BUCKET_SIZE):
        sums = sums + kb[:, t, :]
    rows = [jnp.zeros((1, d_h), jnp.float32)]
    run = jnp.zeros((1, d_h), jnp.float32)
    for j in range(1, buckets):
        run = run + sums[j - 1:j]
        rows.append(run)
    excl = jnp.concatenate(rows, axis=0)
    bidx = jax.lax.broadcasted_iota(jnp.int32, (buckets, 1), 0)
    counts = bidx.astype(jnp.float32) * BUCKET_SIZE + 1.0
    cum_first = (excl + firsts) / counts
    x = jnp.concatenate([cum_first, firsts], axis=-1)

    w = w_ref[0]
    r = jnp.dot(x.astype(jnp.bfloat16), w.astype(jnp.bfloat16),
                preferred_element_type=jnp.float32)
    r = jnp.where(r >= 0, r, 0.01 * r)
    d = r.shape[-1]

    row = jax.lax.broadcasted_iota(jnp.int32, (buckets, d), 0)
    col = jax.lax.broadcasted_iota(jnp.int32, (buckets, d), 1)
    mask_value = -jnp.finfo(r.dtype).max
    xx = jnp.where(col >= row + topk, mask_value, r)

    temp = jnp.float32(TEMPERATURE)
    parts = []
    for ii in range(topk):
        xs = xx / temp
        m = jnp.max(xs, axis=-1, keepdims=True)
        e = jnp.exp(xs - m)
        s = jnp.sum(e, axis=-1, keepdims=True)
        val = 1.0 / s
        cand = jnp.where(xs == m, col, d)
        idx = jnp.min(cand, axis=-1, keepdims=True)
        oh = col == idx
        parts.append(jnp.where(oh, val, 0.0))
        if ii != topk - 1:
            xx = jnp.where(oh, -jnp.inf, xx)
    out = jnp.stack(parts, axis=1)
    out_ref[0] = out.reshape(buckets * topk, d)


def kernel(q, k, linear, topk):
    del q, topk
    bh, seq, d_h = k.shape
    buckets = seq // BUCKET_SIZE
    topk_static = linear.shape[-1] - MAX_BUCKETS
    w = linear[0]

    body = functools.partial(_sortnet_kernel, topk=topk_static, buckets=buckets)
    out = pl.pallas_call(
        body,
        grid=(bh,),
        in_specs=[
            pl.BlockSpec((1, seq, d_h), lambda i: (i, 0, 0)),
            pl.BlockSpec((1, DIM, buckets + topk_static),
                         lambda i: (i % HEADS, 0, 0)),
        ],
        out_specs=pl.BlockSpec(
            (1, buckets * topk_static, buckets + topk_static),
            lambda i: (i, 0, 0)),
        out_shape=jax.ShapeDtypeStruct(
            (bh, buckets * topk_static, buckets + topk_static), jnp.float32),
        compiler_params=pltpu.CompilerParams(
            dimension_semantics=("arbitrary",)),
    )(k, w)
    return out


# speedup vs baseline: 11.2178x; 1.7827x over previous
import functools

import jax
import jax.numpy as jnp
from jax import lax
from jax.experimental import pallas as pl
import jax.experimental.pallas.tpu as pltpu
from jax.experimental.pallas import tpu_sc as plsc

HEADS = 32
BUCKET_SIZE = 128
MAX_BUCKETS = 64
N_TOP_BUCKETS = 8
DIM = 256
TEMPERATURE = 0.75
N_SLICES = 4
PAD_D = 80
SC_ROWS = 64
N_TILES = 32


def _routing_kernel(k_ref, w_ref, xs_ref, excl_ref, *, topk, buckets, n_slices):
    kb = k_ref[...]
    d_h = kb.shape[-1]
    kb = kb.reshape(n_slices, buckets, BUCKET_SIZE, d_h)
    firsts = sums = None
    for g in range(BUCKET_SIZE // 8):
        chunk = kb[:, :, 8 * g:8 * g + 8, :].transpose(2, 0, 1, 3)
        for s in range(8):
            if g == 0 and s == 0:
                firsts = sums = chunk[0]
            else:
                sums = sums + chunk[s]
    run = jnp.zeros((n_slices, 1, d_h), jnp.float32)
    excl_ref[:, 0:1, :] = run
    for j in range(1, buckets):
        run = run + sums[:, j - 1:j]
        excl_ref[:, j:j + 1, :] = run
    excl = excl_ref[...]
    bidx = jax.lax.broadcasted_iota(jnp.int32, (buckets, 1), 0)
    counts = bidx.astype(jnp.float32) * BUCKET_SIZE + 1.0
    cum_first = (excl + firsts) / counts
    x = jnp.concatenate([cum_first, firsts], axis=-1)

    w = w_ref[...]
    r = jnp.concatenate(
        [jnp.dot(x[sl].astype(jnp.bfloat16), w[sl].astype(jnp.bfloat16),
                 preferred_element_type=jnp.float32)[None]
         for sl in range(n_slices)], axis=0)
    r = jnp.where(r >= 0, r, 0.01 * r)
    d = r.shape[-1]

    row = jax.lax.broadcasted_iota(jnp.int32, (n_slices, buckets, d), 1)
    col = jax.lax.broadcasted_iota(jnp.int32, (n_slices, buckets, d), 2)
    mask_value = -jnp.finfo(r.dtype).max
    xx = jnp.where(col >= row + topk, mask_value, r)
    xs = xx / jnp.float32(TEMPERATURE)
    pad = jnp.full((n_slices, buckets, PAD_D - d), -jnp.inf, jnp.float32)
    xsp = jnp.concatenate([xs, pad], axis=-1)
    n_t = (n_slices * buckets) // SC_ROWS
    xst = xsp.reshape(n_t, SC_ROWS, PAD_D).transpose(0, 2, 1)
    xs_ref[...] = xst.reshape(n_t, 1, PAD_D * SC_ROWS)


def _topk_sc_kernel(xs_hbm, out_hbm, xs_v, out_v, *, topk, d):
    wid = lax.axis_index("s") * 2 + lax.axis_index("c")
    pltpu.sync_copy(xs_hbm.at[wid, 0], xs_v)
    lanes = lax.iota(jnp.int32, 16)
    rows_per_b = MAX_BUCKETS * topk

    def _sel_body(gi, carry):
        g = gi // topk
        ii = gi % topk
        lbase = g * 16
        m = jnp.full((16,), -jnp.inf, jnp.float32)
        for c in range(d):
            m = jnp.maximum(m, xs_v[pl.ds(c * SC_ROWS + lbase, 16)])
        s = jnp.zeros((16,), jnp.float32)
        idxv = jnp.full((16,), d, jnp.int32)
        for c in range(d):
            v = xs_v[pl.ds(c * SC_ROWS + lbase, 16)]
            s = s + jnp.exp(v - m)
            idxv = jnp.minimum(idxv, jnp.where(v == m, c, d))
        vals = 1.0 / s
        ninf = jnp.full((16,), -jnp.inf, jnp.float32)
        for c in range(d):
            off = c * SC_ROWS + lbase
            v = xs_v[pl.ds(off, 16)]
            xs_v[pl.ds(off, 16)] = jnp.where(idxv == c, ninf, v)
        for r in range(16):
            l = lbase + r
            sv = vals[r]
            si = idxv[r]
            obase2 = (l * topk + ii) * d
            siv = jnp.full((16,), si, jnp.int32)
            svv = jnp.full((16,), sv, jnp.float32)
            zero = jnp.zeros((16,), jnp.float32)
            for j in range(4):
                cs = lanes + (16 * j)
                out_v[pl.ds(obase2 + 16 * j, 16)] = jnp.where(
                    cs == siv, svv, zero)
            cur = out_v[pl.ds(obase2 + 64, 16)]
            cs = lanes + 64
            keep = jnp.where(lanes < 8, zero, cur)
            out_v[pl.ds(obase2 + 64, 16)] = jnp.where(cs == siv, svv, keep)
        return carry

    n_words = rows_per_b * d
    lax.fori_loop(0, (SC_ROWS // 16) * topk, _sel_body, 0)
    pltpu.sync_copy(out_v.at[pl.ds(0, n_words)],
                    out_hbm.at[pl.ds(wid * n_words, n_words)])


def kernel(q, k, linear, topk):
    del q, topk
    bh, seq, d_h = k.shape
    buckets = seq // BUCKET_SIZE
    topk_static = linear.shape[-1] - MAX_BUCKETS
    d = buckets + topk_static
    w = linear[0]
    ns = N_SLICES
    n_t = (ns * buckets) // SC_ROWS

    body = functools.partial(_routing_kernel, topk=topk_static,
                             buckets=buckets, n_slices=ns)
    sc_body = functools.partial(_topk_sc_kernel, topk=topk_static, d=d)
    chunks = 2
    progs = bh // ns // chunks
    halves = []
    for h in range(chunks):
        xs = pl.pallas_call(
            body,
            grid=(progs,),
            in_specs=[
                pl.BlockSpec((ns, seq, d_h),
                             lambda i, _h=h: (i + _h * progs, 0, 0)),
                pl.BlockSpec((ns, DIM, d),
                             lambda i, _h=h: ((i + _h * progs) % (HEADS // ns),
                                              0, 0)),
            ],
            out_specs=pl.BlockSpec((n_t, 1, PAD_D * SC_ROWS),
                                   lambda i: (i, 0, 0)),
            out_shape=jax.ShapeDtypeStruct(
                (N_TILES, 1, PAD_D * SC_ROWS), jnp.float32),
            scratch_shapes=[pltpu.VMEM((ns, buckets, d_h), jnp.float32)],
            compiler_params=pltpu.CompilerParams(
                dimension_semantics=("arbitrary",)),
        )(k, w)
        n_orows = (bh // chunks) * buckets * topk_static
        out_h = pl.kernel(
            sc_body,
            out_type=jax.ShapeDtypeStruct((n_orows * d,), jnp.float32),
            mesh=plsc.VectorSubcoreMesh(core_axis_name="c",
                                        subcore_axis_name="s"),
            scratch_types=[
                pltpu.VMEM((PAD_D * SC_ROWS,), jnp.float32),
                pltpu.VMEM((MAX_BUCKETS * topk_static * d + 16,), jnp.float32),
            ],
        )(xs)
        halves.append(out_h.reshape(bh // chunks, buckets * topk_static, d))
    return jnp.concatenate(halves, axis=0)
